# baseline (speedup 1.0000x reference)
import jax
import jax.numpy as jnp
from jax.experimental import pallas as pl
from jax.experimental.pallas import tpu as pltpu

MARGIN = 0.3
PD_EPS = 1e-6
B = 1024
D = 2048
BM = 256
NI = B // BM
PC = 256
NC = B // PC
BIG = 3.0e38


def _triplet_kernel(x1_ref, x2_ref, out_ref,
                    pbf_ref, colp_ref, cross_ref, araw_ref, praw_ref,
                    asem, psem):
    k = pl.program_id(0)
    ones_row = jnp.ones((1, D), jnp.float32)

    @pl.when(k == 0)
    def _init():
        pltpu.make_async_copy(
            x1_ref.at[pl.ds(0, BM), 0, :],
            araw_ref.at[0], asem.at[0]).start()
        for q in range(NC):
            pltpu.make_async_copy(
                x2_ref.at[pl.ds(q * PC, PC), 1, :],
                praw_ref.at[q], psem.at[q]).start()
        for q in range(1, NI):
            pltpu.make_async_copy(
                x1_ref.at[pl.ds(q * BM, BM), 0, :],
                araw_ref.at[q], asem.at[q]).start()
        out_ref[...] = jnp.zeros_like(out_ref)
        for c in range(NC):
            pltpu.make_async_copy(
                x2_ref.at[pl.ds(c * PC, PC), 1, :],
                praw_ref.at[c], psem.at[c]).wait()
            praw = praw_ref[c]
            np2 = jax.lax.dot_general(
                praw * praw, ones_row, (((1,), (1,)), ((), ())),
                preferred_element_type=jnp.float32)
            sump = jax.lax.dot_general(
                praw, ones_row, (((1,), (1,)), ((), ())),
                preferred_element_type=jnp.float32)
            t = 1.0 / jnp.maximum(jnp.sqrt(np2), 1e-12)
            pbf_ref[c * PC:(c + 1) * PC, :] = (praw * (2.0 * t)).astype(jnp.bfloat16)
            colp_ref[c * PC:(c + 1) * PC, :] = np2 * t * t - (2.0 * PD_EPS) * sump * t

    @pl.when(k < NI)
    def _matmul_block():
        pltpu.make_async_copy(
            x1_ref.at[pl.ds(k * BM, BM), 0, :],
            araw_ref.at[k], asem.at[k]).wait()
        a = araw_ref[k]
        na2 = jax.lax.dot_general(
            a * a, ones_row, (((1,), (1,)), ((), ())),
            preferred_element_type=jnp.float32)
        ta = 1.0 / jnp.maximum(jnp.sqrt(na2), 1e-12)
        anbf = (a * ta).astype(jnp.bfloat16)
        cross_ref[k % 2] = jax.lax.dot_general(
            pbf_ref[...], anbf, (((1,), (1,)), ((), ())),
            preferred_element_type=jnp.float32)

    @pl.when(k >= 1)
    def _epilogue_block():
        i = k - 1
        h = colp_ref[...] - cross_ref[(k - 1) % 2]

        eye = (jax.lax.broadcasted_iota(jnp.int32, (BM, BM), 0)
               == jax.lax.broadcasted_iota(jnp.int32, (BM, BM), 1))

        hmin = jnp.full((1, BM), BIG, jnp.float32)
        hpos = jnp.zeros((1, BM), jnp.float32)
        for c in range(NC):
            chunk = h[c * BM:(c + 1) * BM, :]
            is_diag = c == i
            m_plain = jnp.min(chunk, axis=0, keepdims=True)
            m_mask = jnp.min(jnp.where(eye, BIG, chunk), axis=0, keepdims=True)
            s_diag = jnp.sum(jnp.where(eye, chunk, 0.0), axis=0, keepdims=True)
            hmin = jnp.minimum(hmin, jnp.where(is_diag, m_mask, m_plain))
            hpos = hpos + jnp.where(is_diag, s_diag, 0.0)

        lv = jnp.maximum(hpos - hmin + MARGIN, 0.0)
        out_ref[...] += jnp.sum(lv, axis=1, keepdims=True) * (1.0 / B)


def kernel(x1, x2):
    out = pl.pallas_call(
        _triplet_kernel,
        grid=(NI + 1,),
        in_specs=[
            pl.BlockSpec(memory_space=pltpu.HBM),
            pl.BlockSpec(memory_space=pltpu.HBM),
        ],
        out_specs=pl.BlockSpec((1, 1), lambda k: (0, 0)),
        out_shape=jax.ShapeDtypeStruct((1, 1), jnp.float32),
        scratch_shapes=[
            pltpu.VMEM((B, D), jnp.bfloat16),
            pltpu.VMEM((B, 1), jnp.float32),
            pltpu.VMEM((2, B, BM), jnp.float32),
            pltpu.VMEM((NI, BM, D), jnp.float32),
            pltpu.VMEM((NC, PC, D), jnp.float32),
            pltpu.SemaphoreType.DMA((NI,)),
            pltpu.SemaphoreType.DMA((NC,)),
        ],
        compiler_params=pltpu.CompilerParams(
            dimension_semantics=("arbitrary",),
        ),
    )(x1, x2)
    return out[0, 0]


# speedup vs baseline: 1.0380x; 1.0380x over previous
import jax
import jax.numpy as jnp
from jax.experimental import pallas as pl
from jax.experimental.pallas import tpu as pltpu

MARGIN = 0.3
PD_EPS = 1e-6
B = 1024
D = 2048
BM = 256
NI = B // BM
PC = 256
NC = B // PC
BIG = 3.0e38


def _eye():
    return (jax.lax.broadcasted_iota(jnp.int32, (BM, BM), 0)
            == jax.lax.broadcasted_iota(jnp.int32, (BM, BM), 1))


def _finish(hpos, hmin, out_ref):
    lv = jnp.maximum(hpos - hmin + MARGIN, 0.0)
    out_ref[...] += jnp.sum(lv, axis=1, keepdims=True) * (1.0 / B)


def _prep_anchor(araw_ref, i, ones_row):
    a = araw_ref[i]
    na2 = jax.lax.dot_general(
        a * a, ones_row, (((1,), (1,)), ((), ())),
        preferred_element_type=jnp.float32)
    ta = 1.0 / jnp.maximum(jnp.sqrt(na2), 1e-12)
    return (a * ta).astype(jnp.bfloat16)


def _triplet_kernel(x1_ref, x2_ref, out_ref,
                    pbf_ref, colp_ref, araw_ref, praw_ref, asem, psem):
    i = pl.program_id(0)
    ones_row = jnp.ones((1, D), jnp.float32)

    @pl.when(i == 0)
    def _init():
        pltpu.make_async_copy(
            x1_ref.at[pl.ds(0, BM), 0, :],
            araw_ref.at[0], asem.at[0]).start()
        for q in range(NC):
            pltpu.make_async_copy(
                x2_ref.at[pl.ds(q * PC, PC), 1, :],
                praw_ref.at[q], psem.at[q]).start()
        for q in range(1, NI):
            pltpu.make_async_copy(
                x1_ref.at[pl.ds(q * BM, BM), 0, :],
                araw_ref.at[q], asem.at[q]).start()
        out_ref[...] = jnp.zeros_like(out_ref)

        pltpu.make_async_copy(
            x1_ref.at[pl.ds(0, BM), 0, :],
            araw_ref.at[0], asem.at[0]).wait()
        anbf = _prep_anchor(araw_ref, 0, ones_row)

        eye = _eye()
        hmin = jnp.full((1, BM), BIG, jnp.float32)
        hpos = jnp.zeros((1, BM), jnp.float32)
        for c in range(NC):
            pltpu.make_async_copy(
                x2_ref.at[pl.ds(c * PC, PC), 1, :],
                praw_ref.at[c], psem.at[c]).wait()
            praw = praw_ref[c]
            np2 = jax.lax.dot_general(
                praw * praw, ones_row, (((1,), (1,)), ((), ())),
                preferred_element_type=jnp.float32)
            sump = jax.lax.dot_general(
                praw, ones_row, (((1,), (1,)), ((), ())),
                preferred_element_type=jnp.float32)
            t = 1.0 / jnp.maximum(jnp.sqrt(np2), 1e-12)
            pc_bf = (praw * (2.0 * t)).astype(jnp.bfloat16)
            colp_c = np2 * t * t - (2.0 * PD_EPS) * sump * t
            pbf_ref[c * PC:(c + 1) * PC, :] = pc_bf
            colp_ref[c * PC:(c + 1) * PC, :] = colp_c

            cross_c = jax.lax.dot_general(
                pc_bf, anbf, (((1,), (1,)), ((), ())),
                preferred_element_type=jnp.float32)
            hc = colp_c - cross_c
            if c == 0:
                hmin = jnp.min(jnp.where(eye, BIG, hc), axis=0, keepdims=True)
                hpos = jnp.sum(jnp.where(eye, hc, 0.0), axis=0, keepdims=True)
            else:
                hmin = jnp.minimum(hmin, jnp.min(hc, axis=0, keepdims=True))
        _finish(hpos, hmin, out_ref)

    @pl.when(i > 0)
    def _block():
        pltpu.make_async_copy(
            x1_ref.at[pl.ds(i * BM, BM), 0, :],
            araw_ref.at[i], asem.at[i]).wait()
        anbf = _prep_anchor(araw_ref, i, ones_row)

        cross = jax.lax.dot_general(
            pbf_ref[...], anbf, (((1,), (1,)), ((), ())),
            preferred_element_type=jnp.float32)
        h = colp_ref[...] - cross

        eye = _eye()
        hmin = jnp.full((1, BM), BIG, jnp.float32)
        hpos = jnp.zeros((1, BM), jnp.float32)
        for c in range(NC):
            chunk = h[c * BM:(c + 1) * BM, :]
            is_diag = c == i
            m_plain = jnp.min(chunk, axis=0, keepdims=True)
            m_mask = jnp.min(jnp.where(eye, BIG, chunk), axis=0, keepdims=True)
            s_diag = jnp.sum(jnp.where(eye, chunk, 0.0), axis=0, keepdims=True)
            hmin = jnp.minimum(hmin, jnp.where(is_diag, m_mask, m_plain))
            hpos = hpos + jnp.where(is_diag, s_diag, 0.0)
        _finish(hpos, hmin, out_ref)


def kernel(x1, x2):
    out = pl.pallas_call(
        _triplet_kernel,
        grid=(NI,),
        in_specs=[
            pl.BlockSpec(memory_space=pltpu.HBM),
            pl.BlockSpec(memory_space=pltpu.HBM),
        ],
        out_specs=pl.BlockSpec((1, 1), lambda i: (0, 0)),
        out_shape=jax.ShapeDtypeStruct((1, 1), jnp.float32),
        scratch_shapes=[
            pltpu.VMEM((B, D), jnp.bfloat16),
            pltpu.VMEM((B, 1), jnp.float32),
            pltpu.VMEM((NI, BM, D), jnp.float32),
            pltpu.VMEM((NC, PC, D), jnp.float32),
            pltpu.SemaphoreType.DMA((NI,)),
            pltpu.SemaphoreType.DMA((NC,)),
        ],
        compiler_params=pltpu.CompilerParams(
            dimension_semantics=("arbitrary",),
        ),
    )(x1, x2)
    return out[0, 0]


# speedup vs baseline: 1.0833x; 1.0437x over previous
import jax
import jax.numpy as jnp
from jax.experimental import pallas as pl
from jax.experimental.pallas import tpu as pltpu

MARGIN = 0.3
PD_EPS = 1e-6
B = 1024
D = 2048
BM = 256
NI = B // BM
PC = 256
NC = B // PC
BIG = 3.0e38


def _triplet_kernel(x1_ref, x2_ref, out_ref,
                    pbf_ref, colp_ref, araw_ref, praw_ref, asem, psem):
    i = pl.program_id(0)
    ones_row = jnp.ones((1, D), jnp.float32)

    @pl.when(i == 0)
    def _init():
        pltpu.make_async_copy(
            x1_ref.at[pl.ds(0, BM), 0, :],
            araw_ref.at[0], asem.at[0]).start()
        for k in range(NC):
            pltpu.make_async_copy(
                x2_ref.at[pl.ds(k * PC, PC), 1, :],
                praw_ref.at[k], psem.at[k]).start()
        for k in range(1, NI):
            pltpu.make_async_copy(
                x1_ref.at[pl.ds(k * BM, BM), 0, :],
                araw_ref.at[k], asem.at[k]).start()
        out_ref[...] = jnp.zeros_like(out_ref)
        for c in range(NC):
            pltpu.make_async_copy(
                x2_ref.at[pl.ds(c * PC, PC), 1, :],
                praw_ref.at[c], psem.at[c]).wait()
            praw = praw_ref[c]
            np2 = jax.lax.dot_general(
                praw * praw, ones_row, (((1,), (1,)), ((), ())),
                preferred_element_type=jnp.float32)
            sump = jax.lax.dot_general(
                praw, ones_row, (((1,), (1,)), ((), ())),
                preferred_element_type=jnp.float32)
            t = 1.0 / jnp.maximum(jnp.sqrt(np2), 1e-12)
            pbf_ref[c * PC:(c + 1) * PC, :] = (praw * (2.0 * t)).astype(jnp.bfloat16)
            colp_ref[c * PC:(c + 1) * PC, :] = np2 * t * t - (2.0 * PD_EPS) * sump * t

    pltpu.make_async_copy(
        x1_ref.at[pl.ds(i * BM, BM), 0, :],
        araw_ref.at[i], asem.at[i]).wait()
    a = araw_ref[i]
    na2 = jax.lax.dot_general(
        a * a, ones_row, (((1,), (1,)), ((), ())),
        preferred_element_type=jnp.float32)
    ta = 1.0 / jnp.maximum(jnp.sqrt(na2), 1e-12)
    anbf = (a * ta).astype(jnp.bfloat16)

    cross = jax.lax.dot_general(
        pbf_ref[...], anbf, (((1,), (1,)), ((), ())),
        preferred_element_type=jnp.float32)
    h = colp_ref[...] - cross

    eye = (jax.lax.broadcasted_iota(jnp.int32, (BM, BM), 0)
           == jax.lax.broadcasted_iota(jnp.int32, (BM, BM), 1))
    eye_big = jnp.where(eye, jnp.float32(BIG), 0.0)

    hmin = jnp.full((1, BM), BIG, jnp.float32)
    hpos = jnp.zeros((1, BM), jnp.float32)
    for k in range(NC):
        chunk = h[k * BM:(k + 1) * BM, :]
        flag = (k == i).astype(jnp.float32)
        m = jnp.min(chunk + eye_big * flag, axis=0, keepdims=True)
        s_diag = jnp.sum(jnp.where(eye, chunk, 0.0), axis=0, keepdims=True)
        hmin = jnp.minimum(hmin, m)
        hpos = hpos + s_diag * flag

    lv = jnp.maximum(hpos - hmin + MARGIN, 0.0)
    out_ref[...] += jnp.sum(lv, axis=1, keepdims=True) * (1.0 / B)


def kernel(x1, x2):
    out = pl.pallas_call(
        _triplet_kernel,
        grid=(NI,),
        in_specs=[
            pl.BlockSpec(memory_space=pltpu.HBM),
            pl.BlockSpec(memory_space=pltpu.HBM),
        ],
        out_specs=pl.BlockSpec((1, 1), lambda i: (0, 0)),
        out_shape=jax.ShapeDtypeStruct((1, 1), jnp.float32),
        scratch_shapes=[
            pltpu.VMEM((B, D), jnp.bfloat16),
            pltpu.VMEM((B, 1), jnp.float32),
            pltpu.VMEM((NI, BM, D), jnp.float32),
            pltpu.VMEM((NC, PC, D), jnp.float32),
            pltpu.SemaphoreType.DMA((NI,)),
            pltpu.SemaphoreType.DMA((NC,)),
        ],
        compiler_params=pltpu.CompilerParams(
            dimension_semantics=("arbitrary",),
        ),
    )(x1, x2)
    return out[0, 0]


# speedup vs baseline: 1.0838x; 1.0005x over previous
import jax
import jax.numpy as jnp
from jax.experimental import pallas as pl
from jax.experimental.pallas import tpu as pltpu

MARGIN = 0.3
PD_EPS = 1e-6
B = 1024
D = 2048
BM = 256
NI = B // BM
PC = 256
NC = B // PC
BIG = 3.0e38


def _triplet_kernel(x1_ref, x2_ref, out_ref,
                    pbf_ref, colp_ref, araw_ref, praw_ref, asem, psem):
    i = pl.program_id(0)
    ones_row = jnp.ones((1, D), jnp.float32)

    @pl.when(i == 0)
    def _init():
        pltpu.make_async_copy(
            x1_ref.at[pl.ds(0, BM), 0, :],
            araw_ref.at[0], asem.at[0]).start()
        for k in range(NC):
            pltpu.make_async_copy(
                x2_ref.at[pl.ds(k * PC, PC), 1, :],
                praw_ref.at[k], psem.at[k]).start()
        for k in range(1, NI):
            pltpu.make_async_copy(
                x1_ref.at[pl.ds(k * BM, BM), 0, :],
                araw_ref.at[k], asem.at[k]).start()
        out_ref[...] = jnp.zeros_like(out_ref)
        for c in range(NC):
            pltpu.make_async_copy(
                x2_ref.at[pl.ds(c * PC, PC), 1, :],
                praw_ref.at[c], psem.at[c]).wait()
            praw = praw_ref[c]
            np2 = jax.lax.dot_general(
                praw * praw, ones_row, (((1,), (1,)), ((), ())),
                preferred_element_type=jnp.float32)
            sump = jax.lax.dot_general(
                praw, ones_row, (((1,), (1,)), ((), ())),
                preferred_element_type=jnp.float32)
            t = 1.0 / jnp.maximum(jnp.sqrt(np2), 1e-12)
            pbf_ref[c * PC:(c + 1) * PC, :] = (praw * (2.0 * t)).astype(jnp.bfloat16)
            colp_ref[c * PC:(c + 1) * PC, :] = np2 * t * t - (2.0 * PD_EPS) * sump * t

    pltpu.make_async_copy(
        x1_ref.at[pl.ds(i * BM, BM), 0, :],
        araw_ref.at[i], asem.at[i]).wait()
    a = araw_ref[i]
    na2 = jax.lax.dot_general(
        a * a, ones_row, (((1,), (1,)), ((), ())),
        preferred_element_type=jnp.float32)
    ta = 1.0 / jnp.maximum(jnp.sqrt(na2), 1e-12)
    anbf = (a * ta).astype(jnp.bfloat16)

    cross = jax.lax.dot_general(
        pbf_ref[...], anbf, (((1,), (1,)), ((), ())),
        preferred_element_type=jnp.float32)

    eye = (jax.lax.broadcasted_iota(jnp.int32, (BM, BM), 0)
           == jax.lax.broadcasted_iota(jnp.int32, (BM, BM), 1))
    eye_big = jnp.where(eye, jnp.float32(BIG), 0.0)

    hmin = jnp.full((1, BM), BIG, jnp.float32)
    hpos = jnp.zeros((1, BM), jnp.float32)
    for k in range(NC):
        chunk = (colp_ref[k * BM:(k + 1) * BM, :]
                 - cross[k * BM:(k + 1) * BM, :])
        flag = (k == i).astype(jnp.float32)
        m = jnp.min(chunk + eye_big * flag, axis=0, keepdims=True)
        s_diag = jnp.sum(jnp.where(eye, chunk, 0.0), axis=0, keepdims=True)
        hmin = jnp.minimum(hmin, m)
        hpos = hpos + s_diag * flag

    lv = jnp.maximum(hpos - hmin + MARGIN, 0.0)
    out_ref[...] += jnp.sum(lv, axis=1, keepdims=True) * (1.0 / B)


def kernel(x1, x2):
    out = pl.pallas_call(
        _triplet_kernel,
        grid=(NI,),
        in_specs=[
            pl.BlockSpec(memory_space=pltpu.HBM),
            pl.BlockSpec(memory_space=pltpu.HBM),
        ],
        out_specs=pl.BlockSpec((1, 1), lambda i: (0, 0)),
        out_shape=jax.ShapeDtypeStruct((1, 1), jnp.float32),
        scratch_shapes=[
            pltpu.VMEM((B, D), jnp.bfloat16),
            pltpu.VMEM((B, 1), jnp.float32),
            pltpu.VMEM((NI, BM, D), jnp.float32),
            pltpu.VMEM((NC, PC, D), jnp.float32),
            pltpu.SemaphoreType.DMA((NI,)),
            pltpu.SemaphoreType.DMA((NC,)),
        ],
        compiler_params=pltpu.CompilerParams(
            dimension_semantics=("arbitrary",),
        ),
    )(x1, x2)
    return out[0, 0]
